# baseline (speedup 1.0000x reference)
import functools

import jax
import jax.numpy as jnp
from jax import lax
from jax.experimental import pallas as pl
from jax.experimental.pallas import tpu as pltpu
from jax.experimental.pallas import tpu_sc as plsc

VOCAB = 100000
EMB = 64
EMBP = 128
HID = 128
B = 1024
L = 50

_NC = 2
_NS = 16
_NW = _NC * _NS
_LC = 25
_BLC = B * _LC
_PER_W = _BLC // _NW
_CHUNK = 100
_NCHUNK = _PER_W // _CHUNK


def _gather_body(table_hbm, idx_hbm, out_hbm, idx_v, rows_v, sem):
    wid = lax.axis_index("s") * _NC + lax.axis_index("c")
    pltpu.sync_copy(idx_hbm.at[pl.ds(wid * _NCHUNK, _NCHUNK)], idx_v)
    copies = []
    for j in range(_NCHUNK):
        copies.append(
            pltpu.async_copy(
                table_hbm.at[idx_v.at[j]],
                rows_v.at[pl.ds(j * _CHUNK, _CHUNK)],
                sem,
            )
        )
    for c in copies:
        c.wait()
    pltpu.sync_copy(rows_v, out_hbm.at[pl.ds(wid * _PER_W, _PER_W)])


@functools.cache
def _make_gather():
    return pl.kernel(
        _gather_body,
        out_type=jax.ShapeDtypeStruct((_BLC, EMBP), jnp.float32),
        mesh=plsc.VectorSubcoreMesh(
            core_axis_name="c", subcore_axis_name="s",
            num_cores=_NC, num_subcores=_NS,
        ),
        scratch_types=[
            pltpu.VMEM((_NCHUNK, _CHUNK), jnp.int32),
            pltpu.VMEM((_PER_W, EMBP), jnp.float32),
            pltpu.SemaphoreType.DMA,
        ],
        compiler_params=pltpu.CompilerParams(use_tc_tiling_on_sc=False),
    )


def _lstm_body(t0, emb_ref, len_ref, h0_ref, c0_ref, wih_ref, whh_ref,
               bih_ref, bhh_ref, h_out, c_out):
    bias = bih_ref[:] + bhh_ref[:]

    def step(t, carry):
        h, c = carry
        xt = emb_ref[t]
        gates = (
            jnp.dot(xt, wih_ref[:], preferred_element_type=jnp.float32)
            + jnp.dot(h, whh_ref[:], preferred_element_type=jnp.float32)
            + bias
        )
        s = 0.5 * jnp.tanh(gates[:, 0 * HID:3 * HID] * 0.5) + 0.5
        i = s[:, 0 * HID:1 * HID]
        f = s[:, 1 * HID:2 * HID]
        o = s[:, 2 * HID:3 * HID]
        g = jnp.tanh(gates[:, 3 * HID:4 * HID])
        c_new = f * c + i * g
        h_new = o * jnp.tanh(c_new)
        m = len_ref[:] > (t + t0)
        return jnp.where(m, h_new, h), jnp.where(m, c_new, c)

    h_last, c_last = lax.fori_loop(0, _LC, step, (h0_ref[:], c0_ref[:]))
    h_out[:] = h_last
    c_out[:] = c_last


_BV = 5000
_NV = VOCAB // _BV


def _proj_body(h_ref, w_ref, b_ref, out_ref):
    out_ref[:] = (
        lax.dot_general(
            w_ref[:], h_ref[:],
            (((1,), (1,)), ((), ())),
            preferred_element_type=jnp.float32,
        )
        + b_ref[:]
    )


def kernel(tokens, lengths, emb_table, W_ih, W_hh, b_ih, b_hh, W_out, b_out):
    tokens = tokens.astype(jnp.int32)
    tok_t = tokens.T
    idx_a = tok_t[:_LC].reshape(_NW * _NCHUNK, _CHUNK)
    idx_b = tok_t[_LC:].reshape(_NW * _NCHUNK, _CHUNK)

    table_p = jnp.concatenate(
        [emb_table.astype(jnp.float32),
         jnp.zeros((VOCAB, EMBP - EMB), jnp.float32)], axis=1)
    gather = _make_gather()
    emb_a = gather(table_p, idx_a).reshape(_LC, B, EMBP)
    emb_b = gather(table_p, idx_b).reshape(_LC, B, EMBP)

    def ifog(w):
        return jnp.concatenate(
            [w[:, :2 * HID], w[:, 3 * HID:], w[:, 2 * HID:3 * HID]], axis=1)

    wih_p = ifog(jnp.pad(W_ih.astype(jnp.float32).T, ((0, EMBP - EMB), (0, 0))))
    whh = ifog(W_hh.astype(jnp.float32).T)
    bih = ifog(b_ih.astype(jnp.float32).reshape(1, 4 * HID))
    bhh = ifog(b_hh.astype(jnp.float32).reshape(1, 4 * HID))
    len2 = lengths.astype(jnp.int32).reshape(B, 1)
    hc_shape = (jax.ShapeDtypeStruct((B, HID), jnp.float32),
                jax.ShapeDtypeStruct((B, HID), jnp.float32))
    zeros = jnp.zeros((B, HID), jnp.float32)

    h1, c1 = pl.pallas_call(
        functools.partial(_lstm_body, 0), out_shape=hc_shape,
    )(emb_a, len2, zeros, zeros, wih_p, whh, bih, bhh)
    h_last, _ = pl.pallas_call(
        functools.partial(_lstm_body, _LC), out_shape=hc_shape,
    )(emb_b, len2, h1, c1, wih_p, whh, bih, bhh)

    logits_t = pl.pallas_call(
        _proj_body,
        grid=(_NV,),
        in_specs=[
            pl.BlockSpec((B, HID), lambda i: (0, 0)),
            pl.BlockSpec((_BV, HID), lambda i: (i, 0)),
            pl.BlockSpec((_BV, 1), lambda i: (i, 0)),
        ],
        out_specs=pl.BlockSpec((_BV, B), lambda i: (i, 0)),
        out_shape=jax.ShapeDtypeStruct((VOCAB, B), jnp.float32),
    )(h_last, W_out.astype(jnp.float32), b_out.astype(jnp.float32).reshape(VOCAB, 1))

    return logits_t.T


# speedup vs baseline: 1.1788x; 1.1788x over previous
import functools

import jax
import jax.numpy as jnp
from jax import lax
from jax.experimental import pallas as pl
from jax.experimental.pallas import tpu as pltpu
from jax.experimental.pallas import tpu_sc as plsc

VOCAB = 100000
EMB = 64
EMBP = 128
HID = 128
B = 1024
L = 50

_NC = 2
_NS = 16
_NW = _NC * _NS
_LC = 25
_BLC = B * _LC
_PER_W = _BLC // _NW
_CHUNK = 100
_NCHUNK = _PER_W // _CHUNK


def _gather_body(table_hbm, idx_hbm, out_hbm, idx_v, rows_v, sem):
    wid = lax.axis_index("s") * _NC + lax.axis_index("c")
    pltpu.sync_copy(idx_hbm.at[pl.ds(wid * _NCHUNK, _NCHUNK)], idx_v)
    copies = []
    for j in range(_NCHUNK):
        copies.append(
            pltpu.async_copy(
                table_hbm.at[idx_v.at[j]],
                rows_v.at[pl.ds(j * _CHUNK, _CHUNK)],
                sem,
            )
        )
    for c in copies:
        c.wait()
    pltpu.sync_copy(rows_v, out_hbm.at[pl.ds(wid * _PER_W, _PER_W)])


@functools.cache
def _make_gather():
    return pl.kernel(
        _gather_body,
        out_type=jax.ShapeDtypeStruct((_BLC, EMBP), jnp.float32),
        mesh=plsc.VectorSubcoreMesh(
            core_axis_name="c", subcore_axis_name="s",
            num_cores=_NC, num_subcores=_NS,
        ),
        scratch_types=[
            pltpu.VMEM((_NCHUNK, _CHUNK), jnp.int32),
            pltpu.VMEM((_PER_W, EMBP), jnp.float32),
            pltpu.SemaphoreType.DMA,
        ],
        compiler_params=pltpu.CompilerParams(use_tc_tiling_on_sc=False),
    )


def _lstm_body(t0, emb_ref, len_ref, h0_ref, c0_ref, wih_ref, whh_ref,
               bih_ref, bhh_ref, h_out, c_out):
    bias = bih_ref[:] + bhh_ref[:]

    def step(t, carry):
        h, c = carry
        xt = emb_ref[t]
        gates = (
            jnp.dot(xt, wih_ref[:], preferred_element_type=jnp.float32)
            + jnp.dot(h, whh_ref[:], preferred_element_type=jnp.float32)
            + bias
        )
        s = 0.5 * jnp.tanh(gates[:, 0 * HID:3 * HID] * 0.5) + 0.5
        i = s[:, 0 * HID:1 * HID]
        f = s[:, 1 * HID:2 * HID]
        o = s[:, 2 * HID:3 * HID]
        g = jnp.tanh(gates[:, 3 * HID:4 * HID])
        c_new = f * c + i * g
        h_new = o * jnp.tanh(c_new)
        m = len_ref[:] > (t + t0)
        return jnp.where(m, h_new, h), jnp.where(m, c_new, c)

    h_last, c_last = lax.fori_loop(0, _LC, step, (h0_ref[:], c0_ref[:]))
    h_out[:] = h_last
    c_out[:] = c_last


_BV = 4096
_NV = (VOCAB + _BV - 1) // _BV


def _proj_body(h_ref, w_ref, b_ref, out_ref):
    bias_col = lax.dot_general(
        b_ref[:], jnp.ones((1, B), jnp.float32),
        (((0,), (0,)), ((), ())),
        preferred_element_type=jnp.float32,
    )
    out_ref[:] = (
        lax.dot_general(
            w_ref[:], h_ref[:],
            (((1,), (1,)), ((), ())),
            preferred_element_type=jnp.float32,
        )
        + bias_col
    )


def kernel(tokens, lengths, emb_table, W_ih, W_hh, b_ih, b_hh, W_out, b_out):
    tokens = tokens.astype(jnp.int32)
    tok_t = tokens.T
    idx_a = tok_t[:_LC].reshape(_NW * _NCHUNK, _CHUNK)
    idx_b = tok_t[_LC:].reshape(_NW * _NCHUNK, _CHUNK)

    table_p = jnp.concatenate(
        [emb_table.astype(jnp.float32),
         jnp.zeros((VOCAB, EMBP - EMB), jnp.float32)], axis=1)
    gather = _make_gather()
    emb_a = gather(table_p, idx_a).reshape(_LC, B, EMBP)
    emb_b = gather(table_p, idx_b).reshape(_LC, B, EMBP)

    def ifog(w):
        return jnp.concatenate(
            [w[:, :2 * HID], w[:, 3 * HID:], w[:, 2 * HID:3 * HID]], axis=1)

    wih_p = ifog(jnp.pad(W_ih.astype(jnp.float32).T, ((0, EMBP - EMB), (0, 0))))
    whh = ifog(W_hh.astype(jnp.float32).T)
    bih = ifog(b_ih.astype(jnp.float32).reshape(1, 4 * HID))
    bhh = ifog(b_hh.astype(jnp.float32).reshape(1, 4 * HID))
    len2 = lengths.astype(jnp.int32).reshape(B, 1)
    hc_shape = (jax.ShapeDtypeStruct((B, HID), jnp.float32),
                jax.ShapeDtypeStruct((B, HID), jnp.float32))
    zeros = jnp.zeros((B, HID), jnp.float32)

    h1, c1 = pl.pallas_call(
        functools.partial(_lstm_body, 0), out_shape=hc_shape,
    )(emb_a, len2, zeros, zeros, wih_p, whh, bih, bhh)
    h_last, _ = pl.pallas_call(
        functools.partial(_lstm_body, _LC), out_shape=hc_shape,
    )(emb_b, len2, h1, c1, wih_p, whh, bih, bhh)

    logits_t = pl.pallas_call(
        _proj_body,
        grid=(_NV,),
        in_specs=[
            pl.BlockSpec((B, HID), lambda i: (0, 0)),
            pl.BlockSpec((_BV, HID), lambda i: (i, 0)),
            pl.BlockSpec((1, _BV), lambda i: (0, i)),
        ],
        out_specs=pl.BlockSpec((_BV, B), lambda i: (i, 0)),
        out_shape=jax.ShapeDtypeStruct((VOCAB, B), jnp.float32),
    )(h_last, W_out.astype(jnp.float32), b_out.astype(jnp.float32).reshape(1, VOCAB))

    return logits_t.T
